# baseline (speedup 1.0000x reference)
import jax
import jax.numpy as jnp
from jax.experimental import pallas as pl
from jax.experimental.pallas import tpu as pltpu

B, N, F_IN, D, C = 64, 128, 128, 128, 2
BG = 8
BF = jnp.bfloat16


def _gnn_body(f1_ref, a1_ref, f2_ref, a2_ref, s1_ref, s2_ref,
              W1_ref, b1_ref, W2_ref, b2_ref, Wa_ref, ba_ref, Wc_ref, bc_ref,
              out_ref):
    W1b = W1_ref[...].astype(BF)
    W2b = W2_ref[...].astype(BF)
    Wab = Wa_ref[...].astype(BF)
    b1 = b1_ref[...]
    b2 = b2_ref[...]
    ba = ba_ref[...]
    iota_row = jax.lax.broadcasted_iota(jnp.int32, (BG, N, 1), 1)
    iota_col = jax.lax.broadcasted_iota(jnp.int32, (1, N), 1)

    def emb(f_ref, a_ref, s_ref):
        sizes = s_ref[...].astype(jnp.float32)
        rowmask = (iota_row.astype(jnp.float32) < sizes[:, :, None]).astype(BF)
        rowmask = rowmask.reshape(BG * N, 1)
        x = f_ref[...].reshape(BG * N, F_IN).astype(BF)
        h = jnp.dot(x, W1b, preferred_element_type=jnp.float32) + b1
        h = h.astype(BF) * rowmask
        As = [a_ref[g].astype(BF) for g in range(BG)]
        sl = lambda g: slice(g * N, (g + 1) * N)
        t1 = [jnp.maximum(
                  jnp.dot(As[g], h[sl(g)],
                          preferred_element_type=jnp.float32).astype(BF), 0)
              for g in range(BG)]
        u = [jnp.dot(t1[g], W2b, preferred_element_type=jnp.float32) + b2
             for g in range(BG)]
        u = [u[g].astype(BF) * rowmask[sl(g)] for g in range(BG)]
        t2 = [jnp.maximum(
                  jnp.dot(As[g], u[g],
                          preferred_element_type=jnp.float32).astype(BF), 0)
              for g in range(BG)]
        cms = [(iota_col.astype(jnp.float32) < sizes[g, 0]).astype(BF)
               for g in range(BG)]
        v = [jnp.dot(cms[g], t2[g], preferred_element_type=jnp.float32)
             for g in range(BG)]
        vcat = jnp.concatenate(v, axis=0).astype(BF)
        e = (jnp.dot(vcat, Wab, preferred_element_type=jnp.float32)
             + sizes * ba)
        return e / jnp.maximum(sizes, 1.0)

    e1 = emb(f1_ref, a1_ref, s1_ref)
    e2 = emb(f2_ref, a2_ref, s2_ref)
    out_ref[...] = (jnp.dot(e1, Wc_ref[:D], preferred_element_type=jnp.float32)
                    + jnp.dot(e2, Wc_ref[D:], preferred_element_type=jnp.float32)
                    + bc_ref[...])


def kernel(feats_1, adjs_1, feats_2, adjs_2, sizes_1, sizes_2,
           W1, b1, W2, b2, Wa, ba, Wc, bc):
    s1 = sizes_1.reshape(B, 1)
    s2 = sizes_2.reshape(B, 1)
    b1r = b1.reshape(1, D)
    b2r = b2.reshape(1, D)
    bar = ba.reshape(1, D)
    bcr = bc.reshape(1, C)

    per_graph = lambda b: (b, 0, 0)
    per_row = lambda b: (b, 0)
    fixed = lambda b: (0, 0)

    out = pl.pallas_call(
        _gnn_body,
        grid=(B // BG,),
        in_specs=[
            pl.BlockSpec((BG, N, F_IN), per_graph),
            pl.BlockSpec((BG, N, N), per_graph),
            pl.BlockSpec((BG, N, F_IN), per_graph),
            pl.BlockSpec((BG, N, N), per_graph),
            pl.BlockSpec((BG, 1), per_row),
            pl.BlockSpec((BG, 1), per_row),
            pl.BlockSpec((F_IN, D), fixed),
            pl.BlockSpec((1, D), fixed),
            pl.BlockSpec((D, D), fixed),
            pl.BlockSpec((1, D), fixed),
            pl.BlockSpec((D, D), fixed),
            pl.BlockSpec((1, D), fixed),
            pl.BlockSpec((2 * D, C), fixed),
            pl.BlockSpec((1, C), fixed),
        ],
        out_specs=pl.BlockSpec((BG, C), per_row),
        out_shape=jax.ShapeDtypeStruct((B, C), jnp.float32),
        compiler_params=pltpu.CompilerParams(
            dimension_semantics=("arbitrary",),
        ),
    )(feats_1, adjs_1, feats_2, adjs_2, s1, s2,
      W1, b1r, W2, b2r, Wa, bar, Wc, bcr)
    return out


# speedup vs baseline: 1.2455x; 1.2455x over previous
import jax
import jax.numpy as jnp
from jax.experimental import pallas as pl
from jax.experimental.pallas import tpu as pltpu

B, N, F_IN, D, C = 64, 128, 128, 128, 2
BG = 16
BF = jnp.bfloat16


def _gnn_body(f1_ref, a1_ref, f2_ref, a2_ref, s1_ref, s2_ref,
              W1_ref, b1_ref, W2_ref, b2_ref, Wa_ref, ba_ref, Wc_ref, bc_ref,
              out_ref):
    W1b = W1_ref[...].astype(BF)
    W2b = W2_ref[...].astype(BF)
    Wab = Wa_ref[...].astype(BF)
    b1 = b1_ref[...]
    b2 = b2_ref[...]
    ba = ba_ref[...]
    iota_row = jax.lax.broadcasted_iota(jnp.int32, (BG, N, 1), 1)
    iota_col = jax.lax.broadcasted_iota(jnp.int32, (1, N), 1)

    def emb(f_ref, a_ref, s_ref):
        sizes = s_ref[...].astype(jnp.float32)
        rowmask = (iota_row.astype(jnp.float32) < sizes[:, :, None]).astype(BF)
        rowmask = rowmask.reshape(BG * N, 1)
        x = f_ref[...].reshape(BG * N, F_IN).astype(BF)
        h = jnp.dot(x, W1b, preferred_element_type=jnp.float32) + b1
        h = h.astype(BF) * rowmask
        As = [a_ref[g].astype(BF) for g in range(BG)]
        sl = lambda g: slice(g * N, (g + 1) * N)
        t1 = [jnp.maximum(
                  jnp.dot(As[g], h[sl(g)],
                          preferred_element_type=jnp.float32).astype(BF), 0)
              for g in range(BG)]
        u = [jnp.dot(t1[g], W2b, preferred_element_type=jnp.float32) + b2
             for g in range(BG)]
        u = [u[g].astype(BF) * rowmask[sl(g)] for g in range(BG)]
        t2 = [jnp.maximum(
                  jnp.dot(As[g], u[g],
                          preferred_element_type=jnp.float32).astype(BF), 0)
              for g in range(BG)]
        cms = [(iota_col.astype(jnp.float32) < sizes[g, 0]).astype(BF)
               for g in range(BG)]
        v = [jnp.dot(cms[g], t2[g], preferred_element_type=jnp.float32)
             for g in range(BG)]
        vcat = jnp.concatenate(v, axis=0).astype(BF)
        e = (jnp.dot(vcat, Wab, preferred_element_type=jnp.float32)
             + sizes * ba)
        return e / jnp.maximum(sizes, 1.0)

    e1 = emb(f1_ref, a1_ref, s1_ref)
    e2 = emb(f2_ref, a2_ref, s2_ref)
    out_ref[...] = (jnp.dot(e1, Wc_ref[:D], preferred_element_type=jnp.float32)
                    + jnp.dot(e2, Wc_ref[D:], preferred_element_type=jnp.float32)
                    + bc_ref[...])


def kernel(feats_1, adjs_1, feats_2, adjs_2, sizes_1, sizes_2,
           W1, b1, W2, b2, Wa, ba, Wc, bc):
    s1 = sizes_1.reshape(B, 1)
    s2 = sizes_2.reshape(B, 1)
    b1r = b1.reshape(1, D)
    b2r = b2.reshape(1, D)
    bar = ba.reshape(1, D)
    bcr = bc.reshape(1, C)

    per_graph = lambda b: (b, 0, 0)
    per_row = lambda b: (b, 0)
    fixed = lambda b: (0, 0)

    out = pl.pallas_call(
        _gnn_body,
        grid=(B // BG,),
        in_specs=[
            pl.BlockSpec((BG, N, F_IN), per_graph),
            pl.BlockSpec((BG, N, N), per_graph),
            pl.BlockSpec((BG, N, F_IN), per_graph),
            pl.BlockSpec((BG, N, N), per_graph),
            pl.BlockSpec((BG, 1), per_row),
            pl.BlockSpec((BG, 1), per_row),
            pl.BlockSpec((F_IN, D), fixed),
            pl.BlockSpec((1, D), fixed),
            pl.BlockSpec((D, D), fixed),
            pl.BlockSpec((1, D), fixed),
            pl.BlockSpec((D, D), fixed),
            pl.BlockSpec((1, D), fixed),
            pl.BlockSpec((2 * D, C), fixed),
            pl.BlockSpec((1, C), fixed),
        ],
        out_specs=pl.BlockSpec((BG, C), per_row),
        out_shape=jax.ShapeDtypeStruct((B, C), jnp.float32),
        compiler_params=pltpu.CompilerParams(
            dimension_semantics=("parallel",),
        ),
    )(feats_1, adjs_1, feats_2, adjs_2, s1, s2,
      W1, b1r, W2, b2r, Wa, bar, Wc, bcr)
    return out
